# baseline (speedup 1.0000x reference)
import functools

import jax
import jax.numpy as jnp
from jax import lax
from jax.experimental import pallas as pl
from jax.experimental.pallas import tpu as pltpu
from jax.experimental.pallas import tpu_sc as plsc

NC = 2
NS = 16
NW = NC * NS
L = 16

B = 16384
E = 64
TROWS = 500000
BPW = B // NW
NCHUNK = 4
CHUNK = BPW // NCHUNK
NBLK = CHUNK // L


def _sc_body(uidx_hbm, cidx_hbm, utab_hbm, ctab_hbm, w_hbm, b_hbm, out_hbm,
             uidx_v, cidx_v, uphys_v, cphys_v, ubuf, cbuf, w_v, b_v, out_v,
             sems):
    c = lax.axis_index("c")
    s = lax.axis_index("s")
    wid = s * NC + c
    base = wid * BPW

    pltpu.sync_copy(uidx_hbm.at[pl.ds(base, BPW)], uidx_v)
    pltpu.sync_copy(cidx_hbm.at[pl.ds(base, BPW)], cidx_v)
    pltpu.sync_copy(w_hbm, w_v)
    pltpu.sync_copy(b_hbm, b_v)

    def phys_body(k, carry):
        sl = pl.ds(k * L, L)
        uphys_v[sl] = lax.shift_right_logical(uidx_v[sl], 1)
        cphys_v[sl] = lax.shift_right_logical(cidx_v[sl], 1)
        return carry
    lax.fori_loop(0, BPW // L, phys_body, 0)

    def fire(j):
        buf = j % 2
        cu = pltpu.async_copy(
            utab_hbm.at[uphys_v.at[pl.ds(j * CHUNK, CHUNK)]], ubuf.at[buf],
            sems.at[buf, 0])
        cc = pltpu.async_copy(
            ctab_hbm.at[cphys_v.at[pl.ds(j * CHUNK, CHUNK)]], cbuf.at[buf],
            sems.at[buf, 1])
        return cu, cc

    wv = w_v[...]
    bv = b_v[...]
    iota = lax.iota(jnp.int32, L)

    def compute(j):
        buf = j % 2
        ub = ubuf.at[buf]
        cb = cbuf.at[buf]

        def blk_body(k, carry):
            row = k * L + iota
            g = pl.ds(j * CHUNK + k * L, L)
            ucol = (uidx_v[g] & 1) << 6
            ccol = (cidx_v[g] & 1) << 6
            acc = jnp.zeros((L,), jnp.float32)
            for d in range(E):
                gu = plsc.load_gather(ub, [row, ucol + d])
                gc = plsc.load_gather(cb, [row, ccol + d])
                acc = acc + gu * gc
            x = acc * wv + bv
            out_v[g] = 1.0 / (1.0 + jnp.exp(-x))
            return carry

        lax.fori_loop(0, NBLK, blk_body, 0)

    cps = {0: fire(0)}
    for j in range(NCHUNK):
        if j + 1 < NCHUNK:
            cps[j + 1] = fire(j + 1)
        cps[j][0].wait()
        cps[j][1].wait()
        compute(j)

    pltpu.sync_copy(out_v, out_hbm.at[pl.ds(base, BPW)])


def _tc_transpose(tabT):
    n = tabT.shape[1]
    blk = 1024
    grid = (n + blk - 1) // blk
    eye = jnp.eye(E, dtype=jnp.float32)

    def body(in_ref, eye_ref, out_ref):
        out_ref[...] = lax.dot_general(
            in_ref[...], eye_ref[...], (((0,), (0,)), ((), ())),
            preferred_element_type=jnp.float32)

    return pl.pallas_call(
        body,
        grid=(grid,),
        compiler_params=pltpu.CompilerParams(
            dimension_semantics=("arbitrary",)),
        in_specs=[pl.BlockSpec((E, blk), lambda i: (0, i)),
                  pl.BlockSpec((E, E), lambda i: (0, 0))],
        out_specs=pl.BlockSpec((blk, E), lambda i: (i, 0)),
        out_shape=jax.ShapeDtypeStruct((n, E), jnp.float32),
    )(tabT, eye)


@jax.jit
def _run(uidx, cidx, user_table, content_table, wvec, bvec):
    mesh = plsc.VectorSubcoreMesh(
        core_axis_name="c", subcore_axis_name="s",
        num_cores=NC, num_subcores=NS)
    return pl.kernel(
        _sc_body,
        out_type=jax.ShapeDtypeStruct((B,), jnp.float32),
        mesh=mesh,
        compiler_params=pltpu.CompilerParams(
            needs_layout_passes=False, use_tc_tiling_on_sc=True),
        scratch_types=[
            pltpu.VMEM((BPW,), jnp.int32),
            pltpu.VMEM((BPW,), jnp.int32),
            pltpu.VMEM((BPW,), jnp.int32),
            pltpu.VMEM((BPW,), jnp.int32),
            pltpu.VMEM((2, CHUNK, 2 * E), jnp.float32),
            pltpu.VMEM((2, CHUNK, 2 * E), jnp.float32),
            pltpu.VMEM((L,), jnp.float32),
            pltpu.VMEM((L,), jnp.float32),
            pltpu.VMEM((BPW,), jnp.float32),
            pltpu.SemaphoreType.DMA((2, 2)),
        ],
    )(uidx, cidx, user_table, content_table, wvec, bvec)


def kernel(inputs, user_table, content_table, dense_w, dense_b):
    uidx = inputs[:, 0]
    cidx = inputs[:, 1]
    utab = _tc_transpose(user_table.T).reshape(TROWS, 2 * E)
    ctab = content_table.reshape(TROWS, 2 * E)
    wvec = jnp.full((L,), dense_w[0, 0], jnp.float32)
    bvec = jnp.full((L,), dense_b[0], jnp.float32)
    out = _run(uidx, cidx, utab, ctab, wvec, bvec)
    return out.reshape(B, 1)


# speedup vs baseline: 1.2814x; 1.2814x over previous
import functools

import jax
import jax.numpy as jnp
from jax import lax
from jax.experimental import pallas as pl
from jax.experimental.pallas import tpu as pltpu
from jax.experimental.pallas import tpu_sc as plsc

NC = 2
NS = 16
NW = NC * NS
L = 16

B = 16384
E = 64
TROWS = 500000
BPW = B // NW
NCHUNK = 4
CHUNK = BPW // NCHUNK
NBLK = CHUNK // L


def _sc_body(uidx_hbm, cidx_hbm, utab_hbm, ctab_hbm, w_hbm, b_hbm, out_hbm,
             uidx_v, cidx_v, uphys_v, cphys_v, ubuf, cbuf, w_v, b_v, out_v,
             sems):
    c = lax.axis_index("c")
    s = lax.axis_index("s")
    wid = s * NC + c
    base = wid * BPW

    pltpu.sync_copy(uidx_hbm.at[pl.ds(base, BPW)], uidx_v)
    pltpu.sync_copy(cidx_hbm.at[pl.ds(base, BPW)], cidx_v)
    pltpu.sync_copy(w_hbm, w_v)
    pltpu.sync_copy(b_hbm, b_v)

    def phys_body(k, carry):
        sl = pl.ds(k * L, L)
        uphys_v[sl] = lax.shift_right_logical(uidx_v[sl], 1)
        cphys_v[sl] = lax.shift_right_logical(cidx_v[sl], 1)
        return carry
    lax.fori_loop(0, BPW // L, phys_body, 0)

    def fire(j):
        buf = j % 2
        cu = pltpu.async_copy(
            utab_hbm.at[uphys_v.at[pl.ds(j * CHUNK, CHUNK)]], ubuf.at[buf],
            sems.at[buf, 0])
        cc = pltpu.async_copy(
            ctab_hbm.at[cphys_v.at[pl.ds(j * CHUNK, CHUNK)]], cbuf.at[buf],
            sems.at[buf, 1])
        return cu, cc

    wv = w_v[...]
    bv = b_v[...]
    iota = lax.iota(jnp.int32, L)

    def compute(j):
        buf = j % 2
        ub = ubuf.at[buf]
        cb = cbuf.at[buf]

        def blk_body(k, carry):
            row = k * L + iota
            g = pl.ds(j * CHUNK + k * L, L)
            ucol = (uidx_v[g] & 1) << 6
            ccol = (cidx_v[g] & 1) << 6
            acc = jnp.zeros((L,), jnp.float32)
            for d in range(E):
                gu = plsc.load_gather(ub, [row, ucol + d])
                gc = plsc.load_gather(cb, [row, ccol + d])
                acc = acc + gu * gc
            x = acc * wv + bv
            out_v[g] = 1.0 / (1.0 + jnp.exp(-x))
            return carry

        lax.fori_loop(0, NBLK, blk_body, 0)

    cps = {0: fire(0)}
    for j in range(NCHUNK):
        if j + 1 < NCHUNK:
            cps[j + 1] = fire(j + 1)
        cps[j][0].wait()
        cps[j][1].wait()
        compute(j)

    pltpu.sync_copy(out_v, out_hbm.at[pl.ds(base, BPW)])


def _tc_transpose(tabT):
    n = tabT.shape[1]
    blk = 4096
    grid = (n + blk - 1) // blk

    def body(in_ref, out_ref):
        out_ref[...] = in_ref[...].T

    return pl.pallas_call(
        body,
        grid=(grid,),
        in_specs=[pl.BlockSpec((E, blk), lambda i: (0, i))],
        out_specs=pl.BlockSpec((blk, E), lambda i: (i, 0)),
        out_shape=jax.ShapeDtypeStruct((n, E), jnp.float32),
    )(tabT)


@jax.jit
def _run(uidx, cidx, user_table, content_table, wvec, bvec):
    mesh = plsc.VectorSubcoreMesh(
        core_axis_name="c", subcore_axis_name="s",
        num_cores=NC, num_subcores=NS)
    return pl.kernel(
        _sc_body,
        out_type=jax.ShapeDtypeStruct((B,), jnp.float32),
        mesh=mesh,
        compiler_params=pltpu.CompilerParams(
            needs_layout_passes=False, use_tc_tiling_on_sc=True),
        scratch_types=[
            pltpu.VMEM((BPW,), jnp.int32),
            pltpu.VMEM((BPW,), jnp.int32),
            pltpu.VMEM((BPW,), jnp.int32),
            pltpu.VMEM((BPW,), jnp.int32),
            pltpu.VMEM((2, CHUNK, 2 * E), jnp.float32),
            pltpu.VMEM((2, CHUNK, 2 * E), jnp.float32),
            pltpu.VMEM((L,), jnp.float32),
            pltpu.VMEM((L,), jnp.float32),
            pltpu.VMEM((BPW,), jnp.float32),
            pltpu.SemaphoreType.DMA((2, 2)),
        ],
    )(uidx, cidx, user_table, content_table, wvec, bvec)


def kernel(inputs, user_table, content_table, dense_w, dense_b):
    uidx = inputs[:, 0]
    cidx = inputs[:, 1]
    utab = _tc_transpose(user_table.T).reshape(TROWS, 2 * E)
    ctab = content_table.reshape(TROWS, 2 * E)
    wvec = jnp.full((L,), dense_w[0, 0], jnp.float32)
    bvec = jnp.full((L,), dense_b[0], jnp.float32)
    out = _run(uidx, cidx, utab, ctab, wvec, bvec)
    return out.reshape(B, 1)


# speedup vs baseline: 1.3076x; 1.0204x over previous
import functools

import jax
import jax.numpy as jnp
from jax import lax
from jax.experimental import pallas as pl
from jax.experimental.pallas import tpu as pltpu
from jax.experimental.pallas import tpu_sc as plsc

NC = 2
NS = 16
NW = NC * NS
L = 16

B = 16384
E = 64
TROWS = 500000
BPW = B // NW
NCHUNK = 4
CHUNK = BPW // NCHUNK
NBLK = CHUNK // L


def _sc_body(uidx_hbm, cidx_hbm, utab_hbm, ctab_hbm, w_hbm, b_hbm, out_hbm,
             uidx_v, cidx_v, uphys_v, cphys_v, ubuf, cbuf, w_v, b_v, out_v,
             sems):
    c = lax.axis_index("c")
    s = lax.axis_index("s")
    wid = s * NC + c
    base = wid * BPW

    pltpu.sync_copy(uidx_hbm.at[pl.ds(base, BPW)], uidx_v)
    pltpu.sync_copy(cidx_hbm.at[pl.ds(base, BPW)], cidx_v)
    pltpu.sync_copy(w_hbm, w_v)
    pltpu.sync_copy(b_hbm, b_v)

    def phys_body(k, carry):
        sl = pl.ds(k * L, L)
        uphys_v[sl] = lax.shift_right_logical(uidx_v[sl], 1)
        cphys_v[sl] = lax.shift_right_logical(cidx_v[sl], 1)
        return carry
    lax.fori_loop(0, BPW // L, phys_body, 0)

    def fire(j):
        buf = j % 2
        cu = pltpu.async_copy(
            utab_hbm.at[uphys_v.at[pl.ds(j * CHUNK, CHUNK)]], ubuf.at[buf],
            sems.at[buf, 0])
        cc = pltpu.async_copy(
            ctab_hbm.at[cphys_v.at[pl.ds(j * CHUNK, CHUNK)]], cbuf.at[buf],
            sems.at[buf, 1])
        return cu, cc

    wv = w_v[...]
    bv = b_v[...]
    iota = lax.iota(jnp.int32, L)

    def compute(j):
        buf = j % 2
        ub = ubuf.at[buf]
        cb = cbuf.at[buf]

        def blk_body(k, carry):
            row = k * L + iota
            g = pl.ds(j * CHUNK + k * L, L)
            ucol = (uidx_v[g] & 1) << 6
            ccol = (cidx_v[g] & 1) << 6
            acc = jnp.zeros((L,), jnp.float32)
            for d in range(E):
                gu = plsc.load_gather(ub, [row, ucol + d])
                gc = plsc.load_gather(cb, [row, ccol + d])
                acc = acc + gu * gc
            x = acc * wv + bv
            out_v[g] = 1.0 / (1.0 + jnp.exp(-x))
            return carry

        lax.fori_loop(0, NBLK, blk_body, 0)

    cps = {0: fire(0)}
    for j in range(NCHUNK):
        if j + 1 < NCHUNK:
            cps[j + 1] = fire(j + 1)
        cps[j][0].wait()
        cps[j][1].wait()
        compute(j)

    pltpu.sync_copy(out_v, out_hbm.at[pl.ds(base, BPW)])


def _tc_transpose(tabT, tail_rm):
    n = tabT.shape[1]
    blk = 2048
    nbuf = 4
    nfull = n // blk
    outer = nfull // nbuf
    assert outer * nbuf == nfull
    tail = n - nfull * blk

    def body(tabT_hbm, tail_hbm, out_hbm, inb, outb, insems, outsems):
        def start_in(g, s):
            pltpu.make_async_copy(
                tabT_hbm.at[:, pl.ds(g * blk, blk)], inb.at[s],
                insems.at[s]).start()

        for s in range(nbuf):
            start_in(jnp.int32(s), s)

        def step(g4, carry):
            for s in range(nbuf):
                g = g4 * nbuf + s
                pltpu.make_async_copy(
                    tabT_hbm.at[:, pl.ds(g * blk, blk)], inb.at[s],
                    insems.at[s]).wait()
                y = inb[s].T

                @pl.when(g4 > 0)
                def _():
                    pltpu.make_async_copy(
                        outb.at[s], out_hbm.at[pl.ds(0, blk)],
                        outsems.at[s]).wait()

                outb[s] = y
                pltpu.make_async_copy(
                    outb.at[s], out_hbm.at[pl.ds(g * blk, blk)],
                    outsems.at[s]).start()

                @pl.when(g + nbuf < nfull)
                def _():
                    start_in(g + nbuf, s)
            return carry

        lax.fori_loop(0, outer, step, 0)
        for s in range(nbuf):
            pltpu.make_async_copy(
                outb.at[s], out_hbm.at[pl.ds(0, blk)], outsems.at[s]).wait()

        t0 = nfull * blk
        pltpu.sync_copy(tail_hbm, outb.at[0, pl.ds(0, tail)])
        pltpu.sync_copy(outb.at[0, pl.ds(0, tail)],
                        out_hbm.at[pl.ds(t0, tail)])

    return pl.pallas_call(
        body,
        in_specs=[pl.BlockSpec(memory_space=pl.ANY),
                  pl.BlockSpec(memory_space=pl.ANY)],
        out_specs=pl.BlockSpec(memory_space=pl.ANY),
        out_shape=jax.ShapeDtypeStruct((n, E), jnp.float32),
        scratch_shapes=[
            pltpu.VMEM((nbuf, E, blk), jnp.float32),
            pltpu.VMEM((nbuf, blk, E), jnp.float32),
            pltpu.SemaphoreType.DMA((nbuf,)),
            pltpu.SemaphoreType.DMA((nbuf,)),
        ],
    )(tabT, tail_rm)


@jax.jit
def _run(uidx, cidx, user_table, content_table, wvec, bvec):
    mesh = plsc.VectorSubcoreMesh(
        core_axis_name="c", subcore_axis_name="s",
        num_cores=NC, num_subcores=NS)
    return pl.kernel(
        _sc_body,
        out_type=jax.ShapeDtypeStruct((B,), jnp.float32),
        mesh=mesh,
        compiler_params=pltpu.CompilerParams(
            needs_layout_passes=False, use_tc_tiling_on_sc=True),
        scratch_types=[
            pltpu.VMEM((BPW,), jnp.int32),
            pltpu.VMEM((BPW,), jnp.int32),
            pltpu.VMEM((BPW,), jnp.int32),
            pltpu.VMEM((BPW,), jnp.int32),
            pltpu.VMEM((2, CHUNK, 2 * E), jnp.float32),
            pltpu.VMEM((2, CHUNK, 2 * E), jnp.float32),
            pltpu.VMEM((L,), jnp.float32),
            pltpu.VMEM((L,), jnp.float32),
            pltpu.VMEM((BPW,), jnp.float32),
            pltpu.SemaphoreType.DMA((2, 2)),
        ],
    )(uidx, cidx, user_table, content_table, wvec, bvec)


def kernel(inputs, user_table, content_table, dense_w, dense_b):
    uidx = inputs[:, 0]
    cidx = inputs[:, 1]
    utab = _tc_transpose(user_table.T,
                         user_table[488 * 2048:, :]).reshape(TROWS, 2 * E)
    ctab = content_table.reshape(TROWS, 2 * E)
    wvec = jnp.full((L,), dense_w[0, 0], jnp.float32)
    bvec = jnp.full((L,), dense_b[0], jnp.float32)
    out = _run(uidx, cidx, utab, ctab, wvec, bvec)
    return out.reshape(B, 1)


# speedup vs baseline: 1.3214x; 1.0106x over previous
import functools

import jax
import jax.numpy as jnp
from jax import lax
from jax.experimental import pallas as pl
from jax.experimental.pallas import tpu as pltpu
from jax.experimental.pallas import tpu_sc as plsc

NC = 2
NS = 16
NW = NC * NS
L = 16

B = 16384
E = 64
TROWS = 500000
BPW = B // NW
NCHUNK = 4
CHUNK = BPW // NCHUNK
NBLK = CHUNK // L


def _sc_body(uidx_hbm, cidx_hbm, utab_hbm, ctab_hbm, w_hbm, b_hbm, out_hbm,
             uidx_v, cidx_v, uphys_v, cphys_v, ubuf, cbuf, w_v, b_v, out_v,
             sems):
    c = lax.axis_index("c")
    s = lax.axis_index("s")
    wid = s * NC + c
    base = wid * BPW

    pltpu.sync_copy(uidx_hbm.at[pl.ds(base, BPW)], uidx_v)
    pltpu.sync_copy(cidx_hbm.at[pl.ds(base, BPW)], cidx_v)
    pltpu.sync_copy(w_hbm, w_v)
    pltpu.sync_copy(b_hbm, b_v)

    def phys_body(k, carry):
        sl = pl.ds(k * L, L)
        uphys_v[sl] = lax.shift_right_logical(uidx_v[sl], 1)
        cphys_v[sl] = lax.shift_right_logical(cidx_v[sl], 1)
        return carry
    lax.fori_loop(0, BPW // L, phys_body, 0)

    def fire(j):
        buf = j % 2
        cu = pltpu.async_copy(
            utab_hbm.at[uphys_v.at[pl.ds(j * CHUNK, CHUNK)]], ubuf.at[buf],
            sems.at[buf, 0])
        cc = pltpu.async_copy(
            ctab_hbm.at[cphys_v.at[pl.ds(j * CHUNK, CHUNK)]], cbuf.at[buf],
            sems.at[buf, 1])
        return cu, cc

    wv = w_v[...]
    bv = b_v[...]
    iota = lax.iota(jnp.int32, L)

    def compute(j):
        buf = j % 2
        ub = ubuf.at[buf]
        cb = cbuf.at[buf]

        def blk_body(k, carry):
            row = k * L + iota
            g = pl.ds(j * CHUNK + k * L, L)
            ucol = (uidx_v[g] & 1) << 6
            ccol = (cidx_v[g] & 1) << 6
            acc = jnp.zeros((L,), jnp.float32)
            for d in range(E):
                gu = plsc.load_gather(ub, [row, ucol + d])
                gc = plsc.load_gather(cb, [row, ccol + d])
                acc = acc + gu * gc
            x = acc * wv + bv
            out_v[g] = 1.0 / (1.0 + jnp.exp(-x))
            return carry

        lax.fori_loop(0, NBLK, blk_body, 0)

    cps = {0: fire(0)}
    for j in range(NCHUNK):
        if j + 1 < NCHUNK:
            cps[j + 1] = fire(j + 1)
        cps[j][0].wait()
        cps[j][1].wait()
        compute(j)

    pltpu.sync_copy(out_v, out_hbm.at[pl.ds(base, BPW)])


def _tc_transpose(tabT, tail_rm):
    n = tabT.shape[1]
    blk = 8192
    nbuf = 2
    nfull = n // blk
    outer = nfull // nbuf
    assert outer * nbuf == nfull
    tail = n - nfull * blk

    def body(tabT_hbm, tail_hbm, out_hbm, inb, outb, insems, outsems):
        ii = lax.broadcasted_iota(jnp.int32, (E, E), 0)
        jj = lax.broadcasted_iota(jnp.int32, (E, E), 1)
        eye = (ii == jj).astype(jnp.float32)

        def start_in(g, s):
            pltpu.make_async_copy(
                tabT_hbm.at[:, pl.ds(g * blk, blk)], inb.at[s],
                insems.at[s]).start()

        for s in range(nbuf):
            start_in(jnp.int32(s), s)

        def step(g4, carry):
            for s in range(nbuf):
                g = g4 * nbuf + s
                pltpu.make_async_copy(
                    tabT_hbm.at[:, pl.ds(g * blk, blk)], inb.at[s],
                    insems.at[s]).wait()
                y = lax.dot_general(
                    inb[s], eye, (((0,), (0,)), ((), ())),
                    preferred_element_type=jnp.float32)

                @pl.when(g4 > 0)
                def _():
                    pltpu.make_async_copy(
                        outb.at[s], out_hbm.at[pl.ds(0, blk)],
                        outsems.at[s]).wait()

                outb[s] = y
                pltpu.make_async_copy(
                    outb.at[s], out_hbm.at[pl.ds(g * blk, blk)],
                    outsems.at[s]).start()

                @pl.when(g + nbuf < nfull)
                def _():
                    start_in(g + nbuf, s)
            return carry

        lax.fori_loop(0, outer, step, 0)
        for s in range(nbuf):
            pltpu.make_async_copy(
                outb.at[s], out_hbm.at[pl.ds(0, blk)], outsems.at[s]).wait()

        t0 = nfull * blk
        pltpu.sync_copy(tail_hbm, outb.at[0, pl.ds(0, tail)])
        pltpu.sync_copy(outb.at[0, pl.ds(0, tail)],
                        out_hbm.at[pl.ds(t0, tail)])

    return pl.pallas_call(
        body,
        compiler_params=pltpu.CompilerParams(skip_device_barrier=True),
        in_specs=[pl.BlockSpec(memory_space=pl.ANY),
                  pl.BlockSpec(memory_space=pl.ANY)],
        out_specs=pl.BlockSpec(memory_space=pl.ANY),
        out_shape=jax.ShapeDtypeStruct((n, E), jnp.float32),
        scratch_shapes=[
            pltpu.VMEM((nbuf, E, blk), jnp.float32),
            pltpu.VMEM((nbuf, blk, E), jnp.float32),
            pltpu.SemaphoreType.DMA((nbuf,)),
            pltpu.SemaphoreType.DMA((nbuf,)),
        ],
    )(tabT, tail_rm)


@jax.jit
def _run(uidx, cidx, user_table, content_table, wvec, bvec):
    mesh = plsc.VectorSubcoreMesh(
        core_axis_name="c", subcore_axis_name="s",
        num_cores=NC, num_subcores=NS)
    return pl.kernel(
        _sc_body,
        out_type=jax.ShapeDtypeStruct((B,), jnp.float32),
        mesh=mesh,
        compiler_params=pltpu.CompilerParams(
            needs_layout_passes=False, use_tc_tiling_on_sc=True),
        scratch_types=[
            pltpu.VMEM((BPW,), jnp.int32),
            pltpu.VMEM((BPW,), jnp.int32),
            pltpu.VMEM((BPW,), jnp.int32),
            pltpu.VMEM((BPW,), jnp.int32),
            pltpu.VMEM((2, CHUNK, 2 * E), jnp.float32),
            pltpu.VMEM((2, CHUNK, 2 * E), jnp.float32),
            pltpu.VMEM((L,), jnp.float32),
            pltpu.VMEM((L,), jnp.float32),
            pltpu.VMEM((BPW,), jnp.float32),
            pltpu.SemaphoreType.DMA((2, 2)),
        ],
    )(uidx, cidx, user_table, content_table, wvec, bvec)


def kernel(inputs, user_table, content_table, dense_w, dense_b):
    uidx = inputs[:, 0]
    cidx = inputs[:, 1]
    utab = _tc_transpose(user_table.T,
                         user_table[488 * 2048:, :]).reshape(TROWS, 2 * E)
    ctab = content_table.reshape(TROWS, 2 * E)
    wvec = jnp.full((L,), dense_w[0, 0], jnp.float32)
    bvec = jnp.full((L,), dense_b[0], jnp.float32)
    out = _run(uidx, cidx, utab, ctab, wvec, bvec)
    return out.reshape(B, 1)


# speedup vs baseline: 1.4351x; 1.0860x over previous
import functools

import jax
import jax.numpy as jnp
from jax import lax
from jax.experimental import pallas as pl
from jax.experimental.pallas import tpu as pltpu
from jax.experimental.pallas import tpu_sc as plsc

NC = 2
NS = 16
NW = NC * NS
L = 16

B = 16384
E = 64
TROWS = 500000
BPW = B // NW
NCHUNK = 4
CHUNK = BPW // NCHUNK

_MESH = dict(core_axis_name="c", subcore_axis_name="s",
             num_cores=NC, num_subcores=NS)
_PARAMS = pltpu.CompilerParams(
    needs_layout_passes=False, use_tc_tiling_on_sc=True)


def _gather_body(idx_hbm, tab_hbm, out_hbm, idx_v, phys_v, buf, sem):
    c = lax.axis_index("c")
    s = lax.axis_index("s")
    wid = s * NC + c
    base = wid * BPW

    pltpu.sync_copy(idx_hbm.at[pl.ds(base, BPW)], idx_v)

    def phys_body(k, carry):
        sl = pl.ds(k * L, L)
        phys_v[sl] = lax.shift_right_logical(idx_v[sl], 1)
        return carry
    lax.fori_loop(0, BPW // L, phys_body, 0)

    def fire(j):
        return pltpu.async_copy(
            tab_hbm.at[phys_v.at[pl.ds(j * CHUNK, CHUNK)]],
            buf.at[j % 2], sem.at[j % 2])

    cps = {0: fire(0)}
    for j in range(NCHUNK):
        if j + 1 < NCHUNK:
            cps[j + 1] = fire(j + 1)
        cps[j].wait()
        pltpu.sync_copy(buf.at[j % 2],
                        out_hbm.at[pl.ds(base + j * CHUNK, CHUNK)])


def _gather_rows(idx, tab):
    return pl.kernel(
        _gather_body,
        out_type=jax.ShapeDtypeStruct((B, 2 * E), jnp.float32),
        mesh=plsc.VectorSubcoreMesh(**_MESH),
        compiler_params=_PARAMS,
        scratch_types=[
            pltpu.VMEM((BPW,), jnp.int32),
            pltpu.VMEM((BPW,), jnp.int32),
            pltpu.VMEM((2, CHUNK, 2 * E), jnp.float32),
            pltpu.SemaphoreType.DMA((2,)),
        ],
    )(idx, tab)


SLAB = 128
NSLAB = BPW // SLAB


def _combine_body(uidx_hbm, cidx_hbm, ug_hbm, cg_hbm, w_hbm, b_hbm, out_hbm,
                  uidx_v, cidx_v, ubuf, cbuf, w_v, b_v, out_v, sems):
    c = lax.axis_index("c")
    s = lax.axis_index("s")
    wid = s * NC + c
    base = wid * BPW

    pltpu.sync_copy(uidx_hbm.at[pl.ds(base, BPW)], uidx_v)
    pltpu.sync_copy(cidx_hbm.at[pl.ds(base, BPW)], cidx_v)
    pltpu.sync_copy(w_hbm, w_v)
    pltpu.sync_copy(b_hbm, b_v)

    def fire(j):
        cu = pltpu.async_copy(
            ug_hbm.at[pl.ds(base + j * SLAB, SLAB)], ubuf.at[j % 2],
            sems.at[j % 2, 0])
        cc = pltpu.async_copy(
            cg_hbm.at[pl.ds(base + j * SLAB, SLAB)], cbuf.at[j % 2],
            sems.at[j % 2, 1])
        return cu, cc

    wv = w_v[...]
    bv = b_v[...]
    iota = lax.iota(jnp.int32, L)

    def compute(j):
        ub = ubuf.at[j % 2]
        cb = cbuf.at[j % 2]

        def blk_body(k, carry):
            row = k * L + iota
            g = pl.ds(j * SLAB + k * L, L)
            ucol = (uidx_v[g] & 1) << 6
            ccol = (cidx_v[g] & 1) << 6
            acc = jnp.zeros((L,), jnp.float32)
            for d in range(E):
                gu = plsc.load_gather(ub, [row, ucol + d])
                gc = plsc.load_gather(cb, [row, ccol + d])
                acc = acc + gu * gc
            x = acc * wv + bv
            out_v[g] = 1.0 / (1.0 + jnp.exp(-x))
            return carry

        lax.fori_loop(0, SLAB // L, blk_body, 0)

    cps = {0: fire(0)}
    for j in range(NSLAB):
        if j + 1 < NSLAB:
            cps[j + 1] = fire(j + 1)
        cps[j][0].wait()
        cps[j][1].wait()
        compute(j)

    pltpu.sync_copy(out_v, out_hbm.at[pl.ds(base, BPW)])


def _combine(uidx, cidx, ug, cg, wvec, bvec):
    return pl.kernel(
        _combine_body,
        out_type=jax.ShapeDtypeStruct((B,), jnp.float32),
        mesh=plsc.VectorSubcoreMesh(**_MESH),
        compiler_params=_PARAMS,
        scratch_types=[
            pltpu.VMEM((BPW,), jnp.int32),
            pltpu.VMEM((BPW,), jnp.int32),
            pltpu.VMEM((2, SLAB, 2 * E), jnp.float32),
            pltpu.VMEM((2, SLAB, 2 * E), jnp.float32),
            pltpu.VMEM((L,), jnp.float32),
            pltpu.VMEM((L,), jnp.float32),
            pltpu.VMEM((BPW,), jnp.float32),
            pltpu.SemaphoreType.DMA((2, 2)),
        ],
    )(uidx, cidx, ug, cg, wvec, bvec)


@jax.jit
def _run(inputs, user_table, content_table, dense_w, dense_b):
    uidx = inputs[:, 0]
    cidx = inputs[:, 1]
    utab = user_table.reshape(TROWS, 2 * E)
    ctab = content_table.reshape(TROWS, 2 * E)
    wvec = jnp.full((L,), dense_w[0, 0], jnp.float32)
    bvec = jnp.full((L,), dense_b[0], jnp.float32)
    ug = _gather_rows(uidx, utab)
    cg = _gather_rows(cidx, ctab)
    out = _combine(uidx, cidx, ug, cg, wvec, bvec)
    return out.reshape(B, 1)


def kernel(inputs, user_table, content_table, dense_w, dense_b):
    return _run(inputs, user_table, content_table, dense_w, dense_b)


# speedup vs baseline: 1.4545x; 1.0135x over previous
import functools

import jax
import jax.numpy as jnp
from jax import lax
from jax.experimental import pallas as pl
from jax.experimental.pallas import tpu as pltpu
from jax.experimental.pallas import tpu_sc as plsc

NC = 2
NS = 16
NW = NC * NS
L = 16

B = 16384
E = 64
TROWS = 500000
BPW = B // NW
NCHUNK = 4
CHUNK = BPW // NCHUNK
NBLK = CHUNK // L


def _sc_body(uidx_hbm, cidx_hbm, utab_hbm, ctab_hbm, w_hbm, b_hbm, out_hbm,
             uidx_v, cidx_v, uphys_v, cphys_v, ubuf, cbuf, w_v, b_v, out_v,
             sems):
    c = lax.axis_index("c")
    s = lax.axis_index("s")
    wid = s * NC + c
    base = wid * BPW

    pltpu.sync_copy(uidx_hbm.at[pl.ds(base, BPW)], uidx_v)
    pltpu.sync_copy(cidx_hbm.at[pl.ds(base, BPW)], cidx_v)
    pltpu.sync_copy(w_hbm, w_v)
    pltpu.sync_copy(b_hbm, b_v)

    def phys_body(k, carry):
        sl = pl.ds(k * L, L)
        uphys_v[sl] = lax.shift_right_logical(uidx_v[sl], 1)
        cphys_v[sl] = lax.shift_right_logical(cidx_v[sl], 1)
        return carry
    lax.fori_loop(0, BPW // L, phys_body, 0)

    def fire(j):
        buf = j % 2
        cu = pltpu.async_copy(
            utab_hbm.at[uphys_v.at[pl.ds(j * CHUNK, CHUNK)]], ubuf.at[buf],
            sems.at[buf, 0])
        cc = pltpu.async_copy(
            ctab_hbm.at[cphys_v.at[pl.ds(j * CHUNK, CHUNK)]], cbuf.at[buf],
            sems.at[buf, 1])
        return cu, cc

    wv = w_v[...]
    bv = b_v[...]
    iota = lax.iota(jnp.int32, L)

    def compute(j):
        buf = j % 2
        ub = ubuf.at[buf]
        cb = cbuf.at[buf]

        def blk_body(k, carry):
            row = k * L + iota
            g = pl.ds(j * CHUNK + k * L, L)
            ucol = (uidx_v[g] & 1) << 6
            ccol = (cidx_v[g] & 1) << 6
            acc = jnp.zeros((L,), jnp.float32)
            for d in range(E):
                gu = plsc.load_gather(ub, [row, ucol + d])
                gc = plsc.load_gather(cb, [row, ccol + d])
                acc = acc + gu * gc
            x = acc * wv + bv
            out_v[g] = 1.0 / (1.0 + jnp.exp(-x))
            return carry

        lax.fori_loop(0, NBLK, blk_body, 0)

    cps = {0: fire(0)}
    for j in range(NCHUNK):
        if j + 1 < NCHUNK:
            cps[j + 1] = fire(j + 1)
        cps[j][0].wait()
        cps[j][1].wait()
        compute(j)

    pltpu.sync_copy(out_v, out_hbm.at[pl.ds(base, BPW)])


@jax.jit
def _run(uidx, cidx, user_table, content_table, wvec, bvec):
    mesh = plsc.VectorSubcoreMesh(
        core_axis_name="c", subcore_axis_name="s",
        num_cores=NC, num_subcores=NS)
    return pl.kernel(
        _sc_body,
        out_type=jax.ShapeDtypeStruct((B,), jnp.float32),
        mesh=mesh,
        compiler_params=pltpu.CompilerParams(
            needs_layout_passes=False, use_tc_tiling_on_sc=True),
        scratch_types=[
            pltpu.VMEM((BPW,), jnp.int32),
            pltpu.VMEM((BPW,), jnp.int32),
            pltpu.VMEM((BPW,), jnp.int32),
            pltpu.VMEM((BPW,), jnp.int32),
            pltpu.VMEM((2, CHUNK, 2 * E), jnp.float32),
            pltpu.VMEM((2, CHUNK, 2 * E), jnp.float32),
            pltpu.VMEM((L,), jnp.float32),
            pltpu.VMEM((L,), jnp.float32),
            pltpu.VMEM((BPW,), jnp.float32),
            pltpu.SemaphoreType.DMA((2, 2)),
        ],
    )(uidx, cidx, user_table, content_table, wvec, bvec)


def kernel(inputs, user_table, content_table, dense_w, dense_b):
    uidx = inputs[:, 0]
    cidx = inputs[:, 1]
    utab = user_table.reshape(TROWS, 2 * E)
    ctab = content_table.reshape(TROWS, 2 * E)
    wvec = jnp.full((L,), dense_w[0, 0], jnp.float32)
    bvec = jnp.full((L,), dense_b[0], jnp.float32)
    out = _run(uidx, cidx, utab, ctab, wvec, bvec)
    return out.reshape(B, 1)
